# baseline (speedup 1.0000x reference)
import jax
import jax.numpy as jnp
from jax import lax
from jax.experimental import pallas as pl
from jax.experimental.pallas import tpu as pltpu
from jax.experimental.pallas import tpu_sc as plsc

B = 8
NS = 100000
R1 = 1600000
R2 = 3200000

NCORES = 2
NSUB = 16
NQ = 4
CHUNK = 320
LANES = 16

R1W = R1 // NQ
R2W = R2 // NQ
N1 = R1W // CHUNK
N2 = R2W // CHUNK


def _body(y_hbm, p1_hbm, p2_hbm, dn_hbm, z_hbm, out_hbm,
          y_v, p_a, p_b, s2_a, s2_b, t_a, t_b,
          acc0, acc1, acc2, acc3,
          lsem_a, lsem_b, ssem_a, ssem_b, ysem):
    c = lax.axis_index("c")
    s = lax.axis_index("s")
    row_local = s // NQ
    q = s % NQ
    accs = (acc0, acc1, acc2, acc3)

    ydma = pltpu.async_copy(y_hbm.at[NQ * c + row_local], y_v, ysem)
    pltpu.sync_copy(dn_hbm, s2_a.at[pl.ds(0, LANES)])
    for rr in range(NQ):
        @pl.when((row_local == rr) & (q == 0))
        def _():
            pltpu.sync_copy(z_hbm, accs[rr])
    dn = plsc.bitcast(s2_a[pl.ds(0, LANES)], jnp.float32)
    ydma.wait()

    plsc.subcore_barrier()

    def scat_fire(t_v, s2_v, ssem):
        for rr in range(NQ):
            @pl.when(row_local == rr)
            def _():
                pltpu.async_copy(t_v, accs[rr].at[s2_v], ssem, add=True)

    def scat_wait(t_v, s2_v, ssem):
        for rr in range(NQ):
            @pl.when(row_local == rr)
            def _():
                pltpu.make_async_copy(t_v, accs[rr].at[s2_v], ssem).wait()

    def fire1(i, p_v, lsem):
        pltpu.async_copy(p1_hbm.at[pl.ds((q * N1 + i) * 3 * CHUNK, 3 * CHUNK)],
                         p_v.at[pl.ds(0, 3 * CHUNK)], lsem)

    def wait1(i, p_v, lsem):
        pltpu.make_async_copy(
            p1_hbm.at[pl.ds((q * N1 + i) * 3 * CHUNK, 3 * CHUNK)],
            p_v.at[pl.ds(0, 3 * CHUNK)], lsem).wait()

    def comp1(p_v, t_v, s2_v):
        def g(j, carry):
            sl = pl.ds(j * LANES, LANES)
            v0 = plsc.load_gather(y_v, [p_v[sl]])
            r = plsc.bitcast(p_v[pl.ds(2 * CHUNK + j * LANES, LANES)],
                             jnp.float32)
            t_v[sl] = r * v0
            s2_v[sl] = p_v[pl.ds(CHUNK + j * LANES, LANES)]
            return carry
        lax.fori_loop(0, CHUNK // LANES, g, 0, unroll=4)

    def fire2(i, p_v, lsem):
        pltpu.async_copy(p2_hbm.at[pl.ds((q * N2 + i) * 4 * CHUNK, 4 * CHUNK)],
                         p_v, lsem)

    def wait2(i, p_v, lsem):
        pltpu.make_async_copy(
            p2_hbm.at[pl.ds((q * N2 + i) * 4 * CHUNK, 4 * CHUNK)],
            p_v, lsem).wait()

    def comp2(p_v, t_v, s2_v):
        def g(j, carry):
            sl = pl.ds(j * LANES, LANES)
            v0 = plsc.load_gather(y_v, [p_v[sl]])
            v1 = plsc.load_gather(y_v, [p_v[pl.ds(CHUNK + j * LANES, LANES)]])
            r = plsc.bitcast(p_v[pl.ds(3 * CHUNK + j * LANES, LANES)],
                             jnp.float32)
            t_v[sl] = (r * dn) * (v0 * v1)
            s2_v[sl] = p_v[pl.ds(2 * CHUNK + j * LANES, LANES)]
            return carry
        lax.fori_loop(0, CHUNK // LANES, g, 0, unroll=4)

    def phase(nchunks, fire, wait, comp):
        nh = nchunks // 2
        fire(0, p_a, lsem_a)

        def body(k, carry):
            fire(2 * k + 1, p_b, lsem_b)
            wait(2 * k, p_a, lsem_a)

            @pl.when(k >= 1)
            def _():
                scat_wait(t_a, s2_a, ssem_a)
            comp(p_a, t_a, s2_a)
            scat_fire(t_a, s2_a, ssem_a)

            @pl.when(k < nh - 1)
            def _():
                fire(2 * k + 2, p_a, lsem_a)
            wait(2 * k + 1, p_b, lsem_b)

            @pl.when(k >= 1)
            def _():
                scat_wait(t_b, s2_b, ssem_b)
            comp(p_b, t_b, s2_b)
            scat_fire(t_b, s2_b, ssem_b)
            return carry
        lax.fori_loop(0, nh, body, 0)
        scat_wait(t_a, s2_a, ssem_a)
        scat_wait(t_b, s2_b, ssem_b)

    phase(N1, fire1, wait1, comp1)
    phase(N2, fire2, wait2, comp2)

    plsc.subcore_barrier()

    for rr in range(NQ):
        @pl.when((row_local == rr) & (q == 0))
        def _():
            pltpu.sync_copy(accs[rr], out_hbm.at[NQ * c + rr])


@jax.jit
def kernel(t_in, y_in, rates_1st, rates_2nd, den_norm, inds_r1, inds_s1,
           inds_r2, inds_s2):
    del t_in
    dn16 = lax.bitcast_convert_type(
        jnp.broadcast_to(den_norm, (LANES,)), jnp.int32)
    r1b = lax.bitcast_convert_type(rates_1st, jnp.int32)
    r2b = lax.bitcast_convert_type(rates_2nd, jnp.int32)
    pack1 = jnp.stack([inds_r1.reshape(-1, CHUNK),
                       inds_s1.reshape(-1, CHUNK),
                       r1b.reshape(-1, CHUNK)], axis=1).reshape(-1)
    pack2 = jnp.stack([inds_r2[:, 0].reshape(-1, CHUNK),
                       inds_r2[:, 1].reshape(-1, CHUNK),
                       inds_s2.reshape(-1, CHUNK),
                       r2b.reshape(-1, CHUNK)], axis=1).reshape(-1)
    zrow = jnp.zeros((NS,), jnp.float32)

    mesh = plsc.VectorSubcoreMesh(core_axis_name="c", subcore_axis_name="s",
                                  num_cores=NCORES, num_subcores=NSUB)
    run = pl.kernel(
        _body,
        out_type=jax.ShapeDtypeStruct((B, NS), jnp.float32),
        mesh=mesh,
        compiler_params=pltpu.CompilerParams(needs_layout_passes=False),
        scratch_types=[
            pltpu.VMEM((NS,), jnp.float32),
            pltpu.VMEM((4 * CHUNK,), jnp.int32),
            pltpu.VMEM((4 * CHUNK,), jnp.int32),
            pltpu.VMEM((CHUNK,), jnp.int32),
            pltpu.VMEM((CHUNK,), jnp.int32),
            pltpu.VMEM((CHUNK,), jnp.float32),
            pltpu.VMEM((CHUNK,), jnp.float32),
            pltpu.VMEM_SHARED((NS,), jnp.float32),
            pltpu.VMEM_SHARED((NS,), jnp.float32),
            pltpu.VMEM_SHARED((NS,), jnp.float32),
            pltpu.VMEM_SHARED((NS,), jnp.float32),
            pltpu.SemaphoreType.DMA,
            pltpu.SemaphoreType.DMA,
            pltpu.SemaphoreType.DMA,
            pltpu.SemaphoreType.DMA,
            pltpu.SemaphoreType.DMA,
        ],
    )
    return run(y_in, pack1, pack2, dn16, zrow)


# speedup vs baseline: 1.4025x; 1.4025x over previous
import jax
import jax.numpy as jnp
from jax import lax
from jax.experimental import pallas as pl
from jax.experimental.pallas import tpu as pltpu
from jax.experimental.pallas import tpu_sc as plsc

B = 8
NS = 100000
R1 = 1600000
R2 = 3200000

NCORES = 2
NSUB = 16
NQ = 4
CHUNK = 320
LANES = 16

R1W = R1 // NQ
R2W = R2 // NQ
N1 = R1W // CHUNK
N2 = R2W // CHUNK


def _body(y_hbm, r1_hbm, r2_hbm, dn_hbm, ir1_hbm, is1_hbm, ir2a_hbm,
          ir2b_hbm, is2_hbm, z_hbm, out_hbm,
          y_v, dn_v, idx_a, idx_b, s_a, s_b, r_a, r_b, s2_a, s2_b, t_a, t_b,
          acc0, acc1, acc2, acc3,
          lsem_a, lsem_b, ssem_a, ssem_b, ysem):
    c = lax.axis_index("c")
    s = lax.axis_index("s")
    row_local = s // NQ
    q = s % NQ
    accs = (acc0, acc1, acc2, acc3)

    ydma = pltpu.async_copy(y_hbm.at[NQ * c + row_local], y_v, ysem)
    pltpu.sync_copy(dn_hbm, dn_v)
    for rr in range(NQ):
        @pl.when((row_local == rr) & (q == 0))
        def _():
            pltpu.sync_copy(z_hbm, accs[rr])
    ydma.wait()
    dn = dn_v[...]

    plsc.subcore_barrier()

    def scat_fire(t_v, s2_v, ssem):
        for rr in range(NQ):
            @pl.when(row_local == rr)
            def _():
                pltpu.async_copy(t_v, accs[rr].at[s2_v], ssem, add=True)

    def scat_wait(t_v, s2_v, ssem):
        for rr in range(NQ):
            @pl.when(row_local == rr)
            def _():
                pltpu.make_async_copy(t_v, accs[rr].at[s2_v], ssem).wait()

    def fire1(i, idx_v, s_v, r_v, lsem):
        base = q * R1W + i * CHUNK
        pltpu.async_copy(ir1_hbm.at[pl.ds(base, CHUNK)],
                         idx_v.at[pl.ds(0, CHUNK)], lsem)
        pltpu.async_copy(is1_hbm.at[pl.ds(base, CHUNK)], s_v, lsem)
        pltpu.async_copy(r1_hbm.at[pl.ds(base, CHUNK)], r_v, lsem)

    def wait1(i, idx_v, s_v, r_v, lsem):
        base = q * R1W + i * CHUNK
        pltpu.make_async_copy(ir1_hbm.at[pl.ds(base, CHUNK)],
                              idx_v.at[pl.ds(0, CHUNK)], lsem).wait()
        pltpu.make_async_copy(is1_hbm.at[pl.ds(base, CHUNK)], s_v, lsem).wait()
        pltpu.make_async_copy(r1_hbm.at[pl.ds(base, CHUNK)], r_v, lsem).wait()

    def comp1(idx_v, s_v, r_v, t_v, s2_v):
        def g(j, carry):
            sl = pl.ds(j * LANES, LANES)
            v0 = plsc.load_gather(y_v, [idx_v[sl]])
            t_v[sl] = r_v[sl] * v0
            s2_v[sl] = s_v[sl]
            return carry
        lax.fori_loop(0, CHUNK // LANES, g, 0, unroll=4)

    def fire2(i, idx_v, s_v, r_v, lsem):
        base = q * R2W + i * CHUNK
        pltpu.async_copy(ir2a_hbm.at[pl.ds(base, CHUNK)],
                         idx_v.at[pl.ds(0, CHUNK)], lsem)
        pltpu.async_copy(ir2b_hbm.at[pl.ds(base, CHUNK)],
                         idx_v.at[pl.ds(CHUNK, CHUNK)], lsem)
        pltpu.async_copy(is2_hbm.at[pl.ds(base, CHUNK)], s_v, lsem)
        pltpu.async_copy(r2_hbm.at[pl.ds(base, CHUNK)], r_v, lsem)

    def wait2(i, idx_v, s_v, r_v, lsem):
        base = q * R2W + i * CHUNK
        pltpu.make_async_copy(ir2a_hbm.at[pl.ds(base, CHUNK)],
                              idx_v.at[pl.ds(0, CHUNK)], lsem).wait()
        pltpu.make_async_copy(ir2b_hbm.at[pl.ds(base, CHUNK)],
                              idx_v.at[pl.ds(CHUNK, CHUNK)], lsem).wait()
        pltpu.make_async_copy(is2_hbm.at[pl.ds(base, CHUNK)], s_v, lsem).wait()
        pltpu.make_async_copy(r2_hbm.at[pl.ds(base, CHUNK)], r_v, lsem).wait()

    def comp2(idx_v, s_v, r_v, t_v, s2_v):
        def g(j, carry):
            sl = pl.ds(j * LANES, LANES)
            v0 = plsc.load_gather(y_v, [idx_v[sl]])
            v1 = plsc.load_gather(y_v, [idx_v[pl.ds(CHUNK + j * LANES, LANES)]])
            t_v[sl] = (r_v[sl] * dn) * (v0 * v1)
            s2_v[sl] = s_v[sl]
            return carry
        lax.fori_loop(0, CHUNK // LANES, g, 0, unroll=4)

    def phase(nchunks, fire, wait, comp):
        nh = nchunks // 2
        fire(0, idx_a, s_a, r_a, lsem_a)

        def body(k, carry):
            fire(2 * k + 1, idx_b, s_b, r_b, lsem_b)
            wait(2 * k, idx_a, s_a, r_a, lsem_a)

            @pl.when(k >= 1)
            def _():
                scat_wait(t_a, s2_a, ssem_a)
            comp(idx_a, s_a, r_a, t_a, s2_a)
            scat_fire(t_a, s2_a, ssem_a)

            @pl.when(k < nh - 1)
            def _():
                fire(2 * k + 2, idx_a, s_a, r_a, lsem_a)
            wait(2 * k + 1, idx_b, s_b, r_b, lsem_b)

            @pl.when(k >= 1)
            def _():
                scat_wait(t_b, s2_b, ssem_b)
            comp(idx_b, s_b, r_b, t_b, s2_b)
            scat_fire(t_b, s2_b, ssem_b)
            return carry
        lax.fori_loop(0, nh, body, 0)
        scat_wait(t_a, s2_a, ssem_a)
        scat_wait(t_b, s2_b, ssem_b)

    phase(N1, fire1, wait1, comp1)
    phase(N2, fire2, wait2, comp2)

    plsc.subcore_barrier()

    for rr in range(NQ):
        @pl.when((row_local == rr) & (q == 0))
        def _():
            pltpu.sync_copy(accs[rr], out_hbm.at[NQ * c + rr])


@jax.jit
def kernel(t_in, y_in, rates_1st, rates_2nd, den_norm, inds_r1, inds_s1,
           inds_r2, inds_s2):
    del t_in
    dn16 = jnp.broadcast_to(den_norm, (LANES,))
    ir2a = inds_r2[:, 0]
    ir2b = inds_r2[:, 1]
    zrow = jnp.zeros((NS,), jnp.float32)

    mesh = plsc.VectorSubcoreMesh(core_axis_name="c", subcore_axis_name="s",
                                  num_cores=NCORES, num_subcores=NSUB)
    run = pl.kernel(
        _body,
        out_type=jax.ShapeDtypeStruct((B, NS), jnp.float32),
        mesh=mesh,
        compiler_params=pltpu.CompilerParams(needs_layout_passes=False),
        scratch_types=[
            pltpu.VMEM((NS,), jnp.float32),
            pltpu.VMEM((LANES,), jnp.float32),
            pltpu.VMEM((2 * CHUNK,), jnp.int32),
            pltpu.VMEM((2 * CHUNK,), jnp.int32),
            pltpu.VMEM((CHUNK,), jnp.int32),
            pltpu.VMEM((CHUNK,), jnp.int32),
            pltpu.VMEM((CHUNK,), jnp.float32),
            pltpu.VMEM((CHUNK,), jnp.float32),
            pltpu.VMEM((CHUNK,), jnp.int32),
            pltpu.VMEM((CHUNK,), jnp.int32),
            pltpu.VMEM((CHUNK,), jnp.float32),
            pltpu.VMEM((CHUNK,), jnp.float32),
            pltpu.VMEM_SHARED((NS,), jnp.float32),
            pltpu.VMEM_SHARED((NS,), jnp.float32),
            pltpu.VMEM_SHARED((NS,), jnp.float32),
            pltpu.VMEM_SHARED((NS,), jnp.float32),
            pltpu.SemaphoreType.DMA,
            pltpu.SemaphoreType.DMA,
            pltpu.SemaphoreType.DMA,
            pltpu.SemaphoreType.DMA,
            pltpu.SemaphoreType.DMA,
        ],
    )
    return run(y_in, rates_1st, rates_2nd, dn16, inds_r1, inds_s1,
               ir2a, ir2b, inds_s2, zrow)
